# Initial kernel scaffold; baseline (speedup 1.0000x reference)
import jax
import jax.numpy as jnp
from jax.experimental import pallas as pl


def kernel(pred_word_fg, pred_word_tblr, pred_word_orient, pred_char_fg, pred_char_tblr, pred_char_cls, im_scale_w, im_scale_h, original_im_w, original_im_h):
    raise NotImplementedError("write your pallas kernel here")


# speedup vs baseline: 1.1456x; 1.1456x over previous
import functools

import jax
import jax.numpy as jnp
from jax import lax
from jax.experimental import pallas as pl
from jax.experimental.pallas import tpu as pltpu

HMAP = 128
WMAP = 128
N = HMAP * WMAP
K = 1024
B_ROWS = 16
A_COLS = 1024

WORD_MIN_SCORE = 0.5
CHAR_MIN_SCORE = 0.25
WORD_NMS_IOU = 0.5
CHAR_NMS_IOU = 0.3
NUM_CHAR_CLASS = 68
STRIDE = 4.0


def _roll(x, s, axis):
    n = x.shape[axis]
    s = s % n
    if s == 0:
        return x
    lo = [slice(None)] * x.ndim
    hi = [slice(None)] * x.ndim
    lo[axis] = slice(s, n)
    hi[axis] = slice(0, s)
    return jnp.concatenate([x[tuple(lo)], x[tuple(hi)]], axis=axis)


def _bitonic_topk(keys, idx, b_io, a_io):
    k = 2
    while k <= N:
        j = k // 2
        while j >= 1:
            if j < A_COLS:
                axis, sh = 2, j
                has_bit = (a_io & j) != 0
            else:
                axis, sh = 1, j >> 10
                has_bit = (b_io & (j >> 10)) != 0
            if k < A_COLS:
                desc = (a_io & k) == 0
            else:
                desc = (b_io & (k >> 10)) == 0
            pk_up = _roll(keys, sh, axis)
            pk_dn = _roll(keys, -sh, axis)
            pi_up = _roll(idx, sh, axis)
            pi_dn = _roll(idx, -sh, axis)
            pk = jnp.where(has_bit, pk_dn, pk_up)
            pi = jnp.where(has_bit, pi_dn, pi_up)
            mine_larger = (keys > pk) | ((keys == pk) & (idx < pi))
            want_larger = desc != has_bit
            take_mine = mine_larger == want_larger
            keys = jnp.where(take_mine, keys, pk)
            idx = jnp.where(take_mine, idx, pi)
            j //= 2
        k *= 2
    return keys, idx


def _transpose(x):
    return x.T


def _gather_feats(stack, topi, n_ch, chunk_ch=8):
    r = topi // WMAP
    c = topi % WMAP
    sub_io = lax.broadcasted_iota(jnp.int32, (HMAP, K), 0)
    row_sel = jnp.where(sub_io == jnp.broadcast_to(r, (HMAP, K)), 1.0, 0.0)
    col_sel = jnp.where(sub_io == jnp.broadcast_to(c, (HMAP, K)), 1.0, 0.0)
    feats = []
    for base in range(0, n_ch, chunk_ch):
        cc = min(chunk_ch, n_ch - base)
        blk = stack[base * HMAP:(base + cc) * HMAP, :]
        h = jnp.dot(blk, col_sel, preferred_element_type=jnp.float32,
                    precision=lax.Precision.HIGHEST)
        for ci in range(cc):
            hm = h[ci * HMAP:(ci + 1) * HMAP, :] * row_sel
            feats.append(jnp.sum(hm, axis=0, keepdims=True))
    return feats


def _nms_and_pack(qx, qy, topv, iou_thresh, wm1, hm1):
    bx1 = jnp.minimum(jnp.minimum(qx[0], qx[1]), jnp.minimum(qx[2], qx[3]))
    bx2 = jnp.maximum(jnp.maximum(qx[0], qx[1]), jnp.maximum(qx[2], qx[3]))
    by1 = jnp.minimum(jnp.minimum(qy[0], qy[1]), jnp.minimum(qy[2], qy[3]))
    by2 = jnp.maximum(jnp.maximum(qy[0], qy[1]), jnp.maximum(qy[2], qy[3]))
    area = (bx2 - bx1) * (by2 - by1)
    validf = topv > 0.0

    cols = jnp.concatenate([bx1, by1, bx2, by2, area, topv], axis=0)
    cols_t = _transpose(cols)
    bx1c = cols_t[:, 0:1]
    by1c = cols_t[:, 1:2]
    bx2c = cols_t[:, 2:3]
    by2c = cols_t[:, 3:4]
    areac = cols_t[:, 4:5]
    validc = cols_t[:, 5:6] > 0.0

    ix1 = jnp.maximum(bx1c, bx1)
    iy1 = jnp.maximum(by1c, by1)
    ix2 = jnp.minimum(bx2c, bx2)
    iy2 = jnp.minimum(by2c, by2)
    inter = jnp.maximum(ix2 - ix1, 0.0) * jnp.maximum(iy2 - iy1, 0.0)
    iou = inter / (areac + area - inter + 1e-6)

    sub_io = lax.broadcasted_iota(jnp.int32, (K, K), 0)
    lane_io = lax.broadcasted_iota(jnp.int32, (K, K), 1)
    sup = (iou > iou_thresh) & (sub_io < lane_io) & validc
    supf = jnp.where(sup, 1.0, 0.0)
    suppressed = jnp.max(supf, axis=0, keepdims=True)
    keepf = jnp.where(validf & (suppressed < 0.5), 1.0, 0.0)

    rows = []
    for i in range(4):
        rows.append(jnp.clip(jnp.round(qx[i]), 0.0, wm1))
        rows.append(jnp.clip(jnp.round(qy[i]), 0.0, hm1))
    rows.append(topv)
    out9 = jnp.concatenate(rows, axis=0) * keepf
    return out9, keepf


def _body(wf_ref, cf_ref, wstack_ref, cstack_ref, params_ref,
          cb_ref, cs_ref, wb_ref):
    wf = wf_ref[...]
    cf = cf_ref[...]
    params = params_ref[...]
    sw = params[:, 0:1]
    sh = params[:, 1:2]
    wm1 = params[:, 2:3]
    hm1 = params[:, 3:4]

    wscore = jnp.where(wf > WORD_MIN_SCORE, wf, 0.0)
    cscore = jnp.where((wf > WORD_MIN_SCORE) & (cf > CHAR_MIN_SCORE), cf, 0.0)

    keys = jnp.stack([wscore, cscore], axis=0)
    b_io = lax.broadcasted_iota(jnp.int32, (2, B_ROWS, A_COLS), 1)
    a_io = lax.broadcasted_iota(jnp.int32, (2, B_ROWS, A_COLS), 2)
    idx = b_io * A_COLS + a_io

    keys, idx = _bitonic_topk(keys, idx, b_io, a_io)

    wtopv = keys[0:1, 0, :]
    wtopi = idx[0:1, 0, :]
    ctopv = keys[1:2, 0, :]
    ctopi = idx[1:2, 0, :]

    sw4 = sw * STRIDE
    sh4 = sh * STRIDE

    wfeats = _gather_feats(wstack_ref[...], wtopi, 5)
    t, b, l, r, orient = wfeats
    xs = (wtopi % WMAP).astype(jnp.float32)
    ys = (wtopi // WMAP).astype(jnp.float32)
    x1 = sw4 * (xs - l)
    y1 = sh4 * (ys - t)
    x2 = sw4 * (xs + r)
    y2 = sh4 * (ys + b)
    cx = sw4 * xs
    cy = sh4 * ys
    cosv = jnp.cos(orient)
    sinv = jnp.sin(orient)
    dx1 = x1 - cx
    dx2 = x2 - cx
    dy1 = y1 - cy
    dy2 = y2 - cy
    qx = [cx + cosv * dx1 - sinv * dy1,
          cx + cosv * dx2 - sinv * dy1,
          cx + cosv * dx2 - sinv * dy2,
          cx + cosv * dx1 - sinv * dy2]
    qy = [cy + sinv * dx1 + cosv * dy1,
          cy + sinv * dx2 + cosv * dy1,
          cy + sinv * dx2 + cosv * dy2,
          cy + sinv * dx1 + cosv * dy2]
    wout9, _ = _nms_and_pack(qx, qy, wtopv, WORD_NMS_IOU, wm1, hm1)
    wb_ref[...] = _transpose(wout9)

    cfeats = _gather_feats(cstack_ref[...], ctopi, 4 + NUM_CHAR_CLASS)
    ct, cb, cl, cr = cfeats[0], cfeats[1], cfeats[2], cfeats[3]
    cls = cfeats[4:]
    cxs = (ctopi % WMAP).astype(jnp.float32)
    cys = (ctopi // WMAP).astype(jnp.float32)
    cx1 = sw4 * (cxs - cl)
    cy1 = sh4 * (cys - ct)
    cx2 = sw4 * (cxs + cr)
    cy2 = sh4 * (cys + cb)
    cqx = [cx1, cx2, cx2, cx1]
    cqy = [cy1, cy1, cy2, cy2]
    cout9, ckeep = _nms_and_pack(cqx, cqy, ctopv, CHAR_NMS_IOU, wm1, hm1)
    cb_ref[...] = _transpose(cout9)
    clsmat = jnp.concatenate(cls, axis=0) * ckeep
    cs_ref[...] = _transpose(clsmat)


@functools.partial(jax.jit, static_argnums=())
def kernel(pred_word_fg, pred_word_tblr, pred_word_orient, pred_char_fg,
           pred_char_tblr, pred_char_cls, im_scale_w, im_scale_h,
           original_im_w, original_im_h):
    wf = pred_word_fg.reshape(B_ROWS, A_COLS)
    cf = pred_char_fg.reshape(B_ROWS, A_COLS)
    wstack = jnp.concatenate(
        [pred_word_tblr, pred_word_orient[None]], axis=0
    ).reshape(5 * HMAP, WMAP)
    cstack = jnp.concatenate(
        [pred_char_tblr, pred_char_cls], axis=0
    ).reshape((4 + NUM_CHAR_CLASS) * HMAP, WMAP)
    wm1 = jnp.asarray(original_im_w, jnp.float32) - 1.0
    hm1 = jnp.asarray(original_im_h, jnp.float32) - 1.0
    params = jnp.stack([
        jnp.asarray(im_scale_w, jnp.float32),
        jnp.asarray(im_scale_h, jnp.float32),
        wm1, hm1,
    ]).reshape(1, 4)

    out_shapes = (
        jax.ShapeDtypeStruct((K, 9), jnp.float32),
        jax.ShapeDtypeStruct((K, NUM_CHAR_CLASS), jnp.float32),
        jax.ShapeDtypeStruct((K, 9), jnp.float32),
    )
    char_bboxes, char_scores, word_bboxes = pl.pallas_call(
        _body,
        out_shape=out_shapes,
    )(wf, cf, wstack, cstack, params)
    return (char_bboxes, char_scores, word_bboxes)
